# baseline (speedup 1.0000x reference)
import jax
import jax.numpy as jnp
from jax import lax
from jax.experimental import pallas as pl
from jax.experimental.pallas import tpu as pltpu
from jax.experimental.pallas import tpu_sc as plsc

B, L, N, D = 4096, 50, 4, 64
P = B * L
NC, NS = 2, 16
NW = NC * NS
P_PER_W = P // NW
CHUNK = 128
IDS = CHUNK * N
NCHUNK = P_PER_W // CHUNK
NGATHER = IDS // 128
LANES = 16


def _body(table_hbm, ids_hbm, out_hbm, gidx_v, rows_v, out_v,
          sem0, sem1, osem0, osem1):
    wid = lax.axis_index("s") * NC + lax.axis_index("c")
    base = wid * P_PER_W
    sems = (sem0, sem1)
    osems = (osem0, osem1)
    iota = lax.iota(jnp.int32, LANES)

    def fire(g, slot):
        pltpu.sync_copy(ids_hbm.at[wid * NCHUNK + g], gidx_v.at[slot])
        for k in range(NGATHER):
            pltpu.async_copy(
                table_hbm.at[gidx_v.at[slot, pl.ds(k * 128, 128)]],
                rows_v.at[slot, pl.ds(k * 128, 128)], sems[slot])

    def drain(slot):
        for k in range(NGATHER):
            pltpu.make_async_copy(
                table_hbm.at[gidx_v.at[slot, pl.ds(k * 128, 128)]],
                rows_v.at[slot, pl.ds(k * 128, 128)], sems[slot]).wait()

    def process(g, slot):
        @pl.when(g >= 2)
        def _():
            pltpu.make_async_copy(
                out_v.at[slot],
                out_hbm.at[pl.ds(base + (g - 2) * CHUNK, CHUNK)],
                osems[slot]).wait()

        drain(slot)

        def grp(s, carry):
            rbase = s * (LANES * N)
            cnt = jnp.zeros((LANES,), jnp.int32)
            for j in range(N):
                ids_j = plsc.load_gather(gidx_v.at[slot],
                                         [rbase + iota * N + j])
                cnt = cnt + jnp.where(ids_j != 0, 1, 0)
            inv = jnp.where(
                cnt == 0, 0.0,
                jnp.where(cnt == 1, 1.0,
                          jnp.where(cnt == 2, 0.5,
                                    jnp.where(cnt == 3, 1.0 / 3.0, 0.25))))
            inv = inv.astype(jnp.float32)
            for i in range(LANES):
                invp = jnp.broadcast_to(inv[i], (LANES,))
                for d in range(D // LANES):
                    dsl = pl.ds(d * LANES, LANES)
                    acc = (rows_v[slot, rbase + 4 * i, dsl]
                           + rows_v[slot, rbase + 4 * i + 1, dsl]
                           + rows_v[slot, rbase + 4 * i + 2, dsl]
                           + rows_v[slot, rbase + 4 * i + 3, dsl])
                    out_v[slot, s * LANES + i, dsl] = acc * invp
            return carry

        lax.fori_loop(0, CHUNK // LANES, grp, 0)
        pltpu.async_copy(out_v.at[slot],
                         out_hbm.at[pl.ds(base + g * CHUNK, CHUNK)],
                         osems[slot])

    fire(0, 0)

    def chunk_pair(it, carry):
        for sub in range(2):
            g = 2 * it + sub

            @pl.when(g + 1 < NCHUNK)
            def _():
                fire(g + 1, 1 - sub)

            process(g, sub)
        return carry

    lax.fori_loop(0, NCHUNK // 2, chunk_pair, 0)

    for slot in range(2):
        g = NCHUNK - 2 + slot
        pltpu.make_async_copy(out_v.at[slot],
                              out_hbm.at[pl.ds(base + g * CHUNK, CHUNK)],
                              osems[slot]).wait()


@jax.jit
def kernel(token_ids, table):
    ids = token_ids.reshape(NW * NCHUNK, IDS)
    mesh = plsc.VectorSubcoreMesh(core_axis_name="c", subcore_axis_name="s")
    out = pl.kernel(
        _body,
        out_type=jax.ShapeDtypeStruct((P, D), jnp.float32),
        mesh=mesh,
        compiler_params=pltpu.CompilerParams(use_tc_tiling_on_sc=False,
                                             needs_layout_passes=False),
        scratch_types=[
            pltpu.VMEM((2, IDS), jnp.int32),
            pltpu.VMEM((2, IDS, D), jnp.float32),
            pltpu.VMEM((2, CHUNK, D), jnp.float32),
            pltpu.SemaphoreType.DMA,
            pltpu.SemaphoreType.DMA,
            pltpu.SemaphoreType.DMA,
            pltpu.SemaphoreType.DMA,
        ],
    )(table, ids)
    return out.reshape(B, L, D)
